# Initial kernel scaffold; baseline (speedup 1.0000x reference)
import jax
import jax.numpy as jnp
from jax.experimental import pallas as pl


def kernel(x, edge_index, edge_attr, batch, We, be, W1, b1, gamma, beta, W2, b2, regW1, regb1, endW, endb):
    raise NotImplementedError("write your pallas kernel here")


# speedup vs baseline: 2.4274x; 2.4274x over previous
import functools

import jax
import jax.numpy as jnp
from jax import lax
from jax.experimental import pallas as pl
from jax.experimental.pallas import tpu as pltpu
from jax.experimental.pallas import tpu_sc as plsc

N = 10000
E = 320000
D = 128
ED = 16
L = 5

NC = 2
NS = 16
NW = NC * NS
EPT = E // NW
CH = 80
NCHUNK = EPT // CH
NP = 10240
RPT = NP // NS
ZR = 128


def _leaky(z):
    return jnp.where(z >= 0, z, 0.01 * z)


def _dot(a, b):
    return jnp.dot(a, b, preferred_element_type=jnp.float32,
                   precision=lax.Precision.HIGHEST)


_EB = 2560


def _edge_proj_body(ea_ref, w_ref, b_ref, o_ref):
    o_ref[...] = _dot(ea_ref[...], w_ref[...]) + b_ref[...]


def _edge_proj(edge_attr, We_l, be_l):
    return pl.pallas_call(
        _edge_proj_body,
        grid=(E // _EB,),
        in_specs=[
            pl.BlockSpec((_EB, ED), lambda i: (i, 0)),
            pl.BlockSpec((ED, D), lambda i: (0, 0)),
            pl.BlockSpec((1, D), lambda i: (0, 0)),
        ],
        out_specs=pl.BlockSpec((_EB, D), lambda i: (i, 0)),
        out_shape=jax.ShapeDtypeStruct((E, D), jnp.float32),
    )(edge_attr, We_l, be_l.reshape(1, D))


_sc_mesh = plsc.VectorSubcoreMesh(core_axis_name="c", subcore_axis_name="s")


@functools.partial(
    pl.kernel,
    out_type=jax.ShapeDtypeStruct((NC, NP, D), jnp.float32),
    mesh=_sc_mesh,
    scratch_types=[
        pltpu.VMEM((CH,), jnp.int32),
        pltpu.VMEM((CH,), jnp.int32),
        pltpu.VMEM((CH, D), jnp.float32),
        pltpu.VMEM((CH, D), jnp.float32),
        pltpu.VMEM((ZR, D), jnp.float32),
        pltpu.VMEM_SHARED((NP, D), jnp.float32),
        pltpu.SemaphoreType.DMA,
    ],
)
def _sc_edge_stage(h_hbm, e_hbm, src_hbm, dst_hbm, out_hbm,
                   srcv, dstv, ebuf, hbuf, zbuf, agg_sh, gsem):
    c = lax.axis_index("c")
    s = lax.axis_index("s")
    wid = s * NC + c
    base = wid * EPT

    @pl.loop(0, ZR)
    def _(r):
        for j in range(D // 16):
            zbuf[r, pl.ds(j * 16, 16)] = jnp.zeros((16,), jnp.float32)

    for k in range(RPT // ZR):
        pltpu.sync_copy(zbuf, agg_sh.at[pl.ds(s * RPT + k * ZR, ZR)])
    plsc.subcore_barrier()

    @pl.loop(0, NCHUNK)
    def _(i):
        off = base + i * CH
        pltpu.sync_copy(src_hbm.at[pl.ds(off, CH)], srcv)
        pltpu.sync_copy(dst_hbm.at[pl.ds(off, CH)], dstv)
        pltpu.sync_copy(e_hbm.at[pl.ds(off, CH)], ebuf)
        pltpu.async_copy(h_hbm.at[srcv], hbuf, gsem).wait()

        @pl.loop(0, CH)
        def _(r):
            for j in range(D // 16):
                sl = pl.ds(j * 16, 16)
                m = hbuf[r, sl] + ebuf[r, sl]
                hbuf[r, sl] = jnp.maximum(m, 0.0)

        pltpu.sync_copy(hbuf, agg_sh.at[dstv], add=True)

    plsc.subcore_barrier()
    pltpu.sync_copy(agg_sh.at[pl.ds(s * RPT, RPT)],
                    out_hbm.at[c].at[pl.ds(s * RPT, RPT)])


def _mlp_body(h_ref, agg_ref, w1_ref, b1_ref, g_ref, bt_ref, w2_ref, b2_ref,
              o_ref):
    z = h_ref[...] + agg_ref[0] + agg_ref[1]
    z = _dot(z, w1_ref[...]) + b1_ref[...]
    mu = jnp.mean(z, axis=0, keepdims=True)
    zc = z - mu
    var = jnp.mean(zc * zc, axis=0, keepdims=True)
    z = zc * lax.rsqrt(var + 1e-5) * g_ref[...] + bt_ref[...]
    z = _leaky(z)
    z = _dot(z, w2_ref[...]) + b2_ref[...]
    o_ref[...] = _leaky(z)


def _mlp(h, agg, W1_l, b1_l, g_l, bt_l, W2_l, b2_l):
    return pl.pallas_call(
        _mlp_body,
        grid=(1,),
        in_specs=[
            pl.BlockSpec((N, D), lambda i: (0, 0)),
            pl.BlockSpec((NC, N, D), lambda i: (0, 0, 0)),
            pl.BlockSpec((D, D), lambda i: (0, 0)),
            pl.BlockSpec((1, D), lambda i: (0, 0)),
            pl.BlockSpec((1, D), lambda i: (0, 0)),
            pl.BlockSpec((1, D), lambda i: (0, 0)),
            pl.BlockSpec((D, D), lambda i: (0, 0)),
            pl.BlockSpec((1, D), lambda i: (0, 0)),
        ],
        out_specs=pl.BlockSpec((N, D), lambda i: (0, 0)),
        out_shape=jax.ShapeDtypeStruct((N, D), jnp.float32),
    )(h, agg, W1_l, b1_l.reshape(1, D), g_l.reshape(1, D),
      bt_l.reshape(1, D), W2_l, b2_l.reshape(1, D))


def _head_body(h_ref, w1_ref, b1_ref, w2_ref, b2_ref, o_ref):
    z = _leaky(_dot(h_ref[...], w1_ref[...]) + b1_ref[...])
    o_ref[...] = _dot(z, w2_ref[...]) + b2_ref[...]


_HB = 2000


def _head(h, regW1, regb1, endW, endb):
    return pl.pallas_call(
        _head_body,
        grid=(N // _HB,),
        in_specs=[
            pl.BlockSpec((_HB, D), lambda i: (i, 0)),
            pl.BlockSpec((D, 500), lambda i: (0, 0)),
            pl.BlockSpec((1, 500), lambda i: (0, 0)),
            pl.BlockSpec((500, 1), lambda i: (0, 0)),
            pl.BlockSpec((1, 1), lambda i: (0, 0)),
        ],
        out_specs=pl.BlockSpec((_HB, 1), lambda i: (i, 0)),
        out_shape=jax.ShapeDtypeStruct((N, 1), jnp.float32),
    )(h, regW1, regb1.reshape(1, -1), endW, endb.reshape(1, 1))


def kernel(x, edge_index, edge_attr, batch, We, be, W1, b1, gamma, beta,
           W2, b2, regW1, regb1, endW, endb):
    src = edge_index[0]
    dst = edge_index[1]
    e_layers = [_edge_proj(edge_attr, We[l], be[l]) for l in range(L)]
    h = x
    for l in range(L):
        agg = _sc_edge_stage(h, e_layers[l], src, dst)
        h = _mlp(h, agg, W1[l], b1[l], gamma[l], beta[l], W2[l], b2[l])
    return _head(h, regW1, regb1, endW, endb)
